# baseline (speedup 1.0000x reference)
import functools

import jax
import jax.numpy as jnp
from jax import lax
from jax.experimental import pallas as pl
from jax.experimental.pallas import tpu as pltpu
from jax.experimental.pallas import tpu_sc as plsc

NG = 8192
D = 32
NS = 16384
BETA = 0.99
WIDTH = 128
XB = 128
NW = 32
ROWS_PER_W = NS // NW
CHUNK = 128
NCHUNK = ROWS_PER_W // CHUNK
STRIPE = NG // 16


GB = 128
NGC = NG // GB


def _assign_body(x_ref, gf_ref, out_ref, gfn_ref):
    i = pl.program_id(0)

    @pl.when(i == 0)
    def _():
        gf = gf_ref[...]
        n = jnp.sqrt(jnp.sum(gf * gf, axis=1, keepdims=True))
        gfn_ref[...] = gf / jnp.maximum(n, 1e-12)

    x = x_ref[...]
    n = jnp.sqrt(jnp.sum(x * x, axis=1, keepdims=True))
    xn = x / jnp.maximum(n, 1e-12)

    m_run = jnp.full((XB, GB), -jnp.inf, jnp.float32)
    c_run = jnp.zeros((XB, GB), jnp.float32)
    for c in range(NGC):
        chunk = lax.dot_general(
            xn, gfn_ref[c * GB:(c + 1) * GB, :], (((1,), (1,)), ((), ())),
            preferred_element_type=jnp.float32)
        better = chunk > m_run
        c_run = jnp.where(better, jnp.float32(c), c_run)
        m_run = jnp.maximum(chunk, m_run)

    m = jnp.max(m_run, axis=-1, keepdims=True)
    lane = lax.broadcasted_iota(jnp.int32, (XB, GB), 1).astype(jnp.float32)
    j = c_run * GB + lane
    loc = jnp.min(jnp.where(m_run == m, j, jnp.float32(NG)), axis=-1)
    out_ref[0, 0, :] = loc.astype(jnp.int32)


def _assign(x, gf):
    grid = NS // XB
    return pl.pallas_call(
        _assign_body,
        grid=(grid,),
        in_specs=[
            pl.BlockSpec((XB, D), lambda i: (i, 0)),
            pl.BlockSpec((NG, D), lambda i: (0, 0)),
        ],
        out_specs=pl.BlockSpec((1, 1, XB), lambda i: (i, 0, 0)),
        out_shape=jax.ShapeDtypeStruct((grid, 1, XB), jnp.int32),
        scratch_shapes=[pltpu.VMEM((NG, D), jnp.float32)],
    )(x, gf)


@functools.lru_cache(maxsize=1)
def _make_scatter():
    mesh = plsc.VectorSubcoreMesh(core_axis_name="c", subcore_axis_name="s")

    @functools.partial(
        pl.kernel,
        mesh=mesh,
        out_type=jax.ShapeDtypeStruct((2 * NG, WIDTH), jnp.float32),
        scratch_types=[
            pltpu.VMEM((NCHUNK, CHUNK), jnp.int32),
            pltpu.VMEM((CHUNK, WIDTH), jnp.float32),
            pltpu.VMEM_SHARED((NG, WIDTH), jnp.float32),
        ],
    )
    def scatter(xaug_hbm, asn_hbm, out_hbm, idx_v, chunk_v, table_sh):
        c = lax.axis_index("c")
        s = lax.axis_index("s")
        wid = s * 2 + c
        base = wid * ROWS_PER_W

        zv = jnp.zeros((16,), jnp.float32)

        def zrow(r, carry):
            for k in range(WIDTH // 16):
                chunk_v[r, pl.ds(k * 16, 16)] = zv
            return carry

        lax.fori_loop(0, CHUNK, zrow, 0)
        for q in range(STRIPE // CHUNK):
            pltpu.sync_copy(chunk_v, table_sh.at[pl.ds(s * STRIPE + q * CHUNK, CHUNK)])
        pltpu.sync_copy(asn_hbm.at[pl.ds(wid * NCHUNK, NCHUNK)], idx_v)
        plsc.subcore_barrier()

        for q in range(NCHUNK):
            pltpu.sync_copy(xaug_hbm.at[pl.ds(base + q * CHUNK, CHUNK)], chunk_v)
            pltpu.sync_copy(chunk_v, table_sh.at[idx_v.at[q]], add=True)
        plsc.subcore_barrier()

        for q in range(STRIPE // CHUNK):
            pltpu.sync_copy(table_sh.at[pl.ds(s * STRIPE + q * CHUNK, CHUNK)], chunk_v)
            pltpu.sync_copy(
                chunk_v,
                out_hbm.at[pl.ds(c * NG + s * STRIPE + q * CHUNK, CHUNK)])

    return scatter


def _combine_body(gf_ref, t_ref, out_ref):
    gf = gf_ref[...]
    t = t_ref[0] + t_ref[1]
    sums = t[:, :D]
    counts = t[:, D:D + 1]
    upd = BETA * gf + (1.0 - BETA) * sums / jnp.maximum(counts, 1.0)
    g = jnp.where(counts > 0, upd, gf)
    n = jnp.sqrt(jnp.sum(g * g, axis=1, keepdims=True))
    out_ref[...] = g / jnp.maximum(n, 1e-12)


def _combine(gf, table):
    return pl.pallas_call(
        _combine_body,
        out_shape=jax.ShapeDtypeStruct((NG, D), jnp.float32),
    )(gf, table)


def kernel(x, group_features):
    asn = _assign(x, group_features).reshape(NW * NCHUNK, CHUNK)
    xaug = jnp.concatenate(
        [x, jnp.ones((NS, 1), jnp.float32),
         jnp.zeros((NS, WIDTH - D - 1), jnp.float32)], axis=1)
    table = _make_scatter()(xaug, asn).reshape(2, NG, WIDTH)
    return _combine(group_features, table)


# speedup vs baseline: 1.8707x; 1.8707x over previous
import functools

import jax
import jax.numpy as jnp
from jax import lax
from jax.experimental import pallas as pl
from jax.experimental.pallas import tpu as pltpu
from jax.experimental.pallas import tpu_sc as plsc

NG = 8192
D = 32
NS = 16384
BETA = 0.99
WIDTH = 128
XB = 512
NW = 32
ROWS_PER_W = NS // NW
CHUNK = 128
NCHUNK = ROWS_PER_W // CHUNK
STRIPE = NG // 16


GB = 128
NGC = NG // GB


def _normalize_body(a_ref, out_ref):
    a = a_ref[...]
    n = jnp.sqrt(jnp.sum(a * a, axis=1, keepdims=True))
    out_ref[...] = a / jnp.maximum(n, 1e-12)


def _normalize_rows(a):
    return pl.pallas_call(
        _normalize_body,
        out_shape=jax.ShapeDtypeStruct(a.shape, jnp.float32),
    )(a)


def _assign_body(x_ref, gfn_ref, out_ref):
    x = x_ref[...]
    n = jnp.sqrt(jnp.sum(x * x, axis=1, keepdims=True))
    xn = x / jnp.maximum(n, 1e-12)

    m_run = jnp.full((XB, GB), -jnp.inf, jnp.float32)
    c_run = jnp.zeros((XB, GB), jnp.float32)
    for c in range(NGC):
        chunk = lax.dot_general(
            xn, gfn_ref[c * GB:(c + 1) * GB, :], (((1,), (1,)), ((), ())),
            preferred_element_type=jnp.float32)
        better = chunk > m_run
        c_run = jnp.where(better, jnp.float32(c), c_run)
        m_run = jnp.maximum(chunk, m_run)

    m = jnp.max(m_run, axis=-1, keepdims=True)
    lane = lax.broadcasted_iota(jnp.int32, (XB, GB), 1).astype(jnp.float32)
    j = c_run * GB + lane
    loc = jnp.min(jnp.where(m_run == m, j, jnp.float32(NG)), axis=-1)
    out_ref[0, 0, :] = loc.astype(jnp.int32)


def _assign(x, gfn):
    grid = NS // XB
    return pl.pallas_call(
        _assign_body,
        grid=(grid,),
        in_specs=[
            pl.BlockSpec((XB, D), lambda i: (i, 0)),
            pl.BlockSpec((NG, D), lambda i: (0, 0)),
        ],
        out_specs=pl.BlockSpec((1, 1, XB), lambda i: (i, 0, 0)),
        out_shape=jax.ShapeDtypeStruct((grid, 1, XB), jnp.int32),
    )(x, gfn)


@functools.lru_cache(maxsize=1)
def _make_scatter():
    mesh = plsc.VectorSubcoreMesh(core_axis_name="c", subcore_axis_name="s")

    @functools.partial(
        pl.kernel,
        mesh=mesh,
        out_type=jax.ShapeDtypeStruct((2 * NG, WIDTH), jnp.float32),
        scratch_types=[
            pltpu.VMEM((NCHUNK, CHUNK), jnp.int32),
            pltpu.VMEM((CHUNK, WIDTH), jnp.float32),
            pltpu.VMEM_SHARED((NG, WIDTH), jnp.float32),
        ],
    )
    def scatter(xaug_hbm, asn_hbm, out_hbm, idx_v, chunk_v, table_sh):
        c = lax.axis_index("c")
        s = lax.axis_index("s")
        wid = s * 2 + c
        base = wid * ROWS_PER_W

        zv = jnp.zeros((16,), jnp.float32)

        def zrow(r, carry):
            for k in range(WIDTH // 16):
                chunk_v[r, pl.ds(k * 16, 16)] = zv
            return carry

        lax.fori_loop(0, CHUNK, zrow, 0)
        for q in range(STRIPE // CHUNK):
            pltpu.sync_copy(chunk_v, table_sh.at[pl.ds(s * STRIPE + q * CHUNK, CHUNK)])
        pltpu.sync_copy(asn_hbm.at[pl.ds(wid * NCHUNK, NCHUNK)], idx_v)
        plsc.subcore_barrier()

        for q in range(NCHUNK):
            pltpu.sync_copy(xaug_hbm.at[pl.ds(base + q * CHUNK, CHUNK)], chunk_v)
            pltpu.sync_copy(chunk_v, table_sh.at[idx_v.at[q]], add=True)
        plsc.subcore_barrier()

        for q in range(STRIPE // CHUNK):
            pltpu.sync_copy(table_sh.at[pl.ds(s * STRIPE + q * CHUNK, CHUNK)], chunk_v)
            pltpu.sync_copy(
                chunk_v,
                out_hbm.at[pl.ds(c * NG + s * STRIPE + q * CHUNK, CHUNK)])

    return scatter


def _combine_body(gf_ref, t_ref, out_ref):
    gf = gf_ref[...]
    t = t_ref[0] + t_ref[1]
    sums = t[:, :D]
    counts = t[:, D:D + 1]
    upd = BETA * gf + (1.0 - BETA) * sums / jnp.maximum(counts, 1.0)
    g = jnp.where(counts > 0, upd, gf)
    n = jnp.sqrt(jnp.sum(g * g, axis=1, keepdims=True))
    out_ref[...] = g / jnp.maximum(n, 1e-12)


def _combine(gf, table):
    return pl.pallas_call(
        _combine_body,
        out_shape=jax.ShapeDtypeStruct((NG, D), jnp.float32),
    )(gf, table)


def kernel(x, group_features):
    gfn = _normalize_rows(group_features)
    asn = _assign(x, gfn).reshape(NW * NCHUNK, CHUNK)
    xaug = jnp.concatenate(
        [x, jnp.ones((NS, 1), jnp.float32),
         jnp.zeros((NS, WIDTH - D - 1), jnp.float32)], axis=1)
    table = _make_scatter()(xaug, asn).reshape(2, NG, WIDTH)
    return _combine(group_features, table)


# speedup vs baseline: 2.0335x; 1.0870x over previous
import functools

import jax
import jax.numpy as jnp
from jax import lax
from jax.experimental import pallas as pl
from jax.experimental.pallas import tpu as pltpu
from jax.experimental.pallas import tpu_sc as plsc

NG = 8192
D = 32
NS = 16384
BETA = 0.99
WIDTH = 128
XB = 512
NW = 32
ROWS_PER_W = NS // NW
CHUNK = 128
NCHUNK = ROWS_PER_W // CHUNK
STRIPE = NG // 16


GB = 128
NGC = NG // GB


def _normalize_body(a_ref, out_ref):
    a = a_ref[...]
    n = jnp.sqrt(jnp.sum(a * a, axis=1, keepdims=True))
    out_ref[...] = a / jnp.maximum(n, 1e-12)


def _normalize_rows(a):
    return pl.pallas_call(
        _normalize_body,
        out_shape=jax.ShapeDtypeStruct(a.shape, jnp.float32),
    )(a)


def _assign_body(x_ref, gfn_ref, out_ref, xaug_ref):
    x = x_ref[...]
    xaug_ref[...] = jnp.concatenate(
        [x, jnp.ones((XB, 1), jnp.float32),
         jnp.zeros((XB, WIDTH - D - 1), jnp.float32)], axis=1)
    n = jnp.sqrt(jnp.sum(x * x, axis=1, keepdims=True))
    xn = x / jnp.maximum(n, 1e-12)

    m_run = jnp.full((XB, GB), -jnp.inf, jnp.float32)
    c_run = jnp.zeros((XB, GB), jnp.float32)
    for c in range(NGC):
        chunk = lax.dot_general(
            xn, gfn_ref[c * GB:(c + 1) * GB, :], (((1,), (1,)), ((), ())),
            preferred_element_type=jnp.float32)
        better = chunk > m_run
        c_run = jnp.where(better, jnp.float32(c), c_run)
        m_run = jnp.maximum(chunk, m_run)

    m = jnp.max(m_run, axis=-1, keepdims=True)
    lane = lax.broadcasted_iota(jnp.int32, (XB, GB), 1).astype(jnp.float32)
    j = c_run * GB + lane
    loc = jnp.min(jnp.where(m_run == m, j, jnp.float32(NG)), axis=-1)
    out_ref[0] = loc.astype(jnp.int32).reshape(XB // CHUNK, CHUNK)


def _assign(x, gfn):
    grid = NS // XB
    rpb = XB // CHUNK
    return pl.pallas_call(
        _assign_body,
        grid=(grid,),
        in_specs=[
            pl.BlockSpec((XB, D), lambda i: (i, 0)),
            pl.BlockSpec((NG, D), lambda i: (0, 0)),
        ],
        out_specs=[
            pl.BlockSpec((1, rpb, CHUNK), lambda i: (i, 0, 0)),
            pl.BlockSpec((XB, WIDTH), lambda i: (i, 0)),
        ],
        out_shape=[
            jax.ShapeDtypeStruct((grid, rpb, CHUNK), jnp.int32),
            jax.ShapeDtypeStruct((NS, WIDTH), jnp.float32),
        ],
    )(x, gfn)


@functools.lru_cache(maxsize=1)
def _make_scatter():
    mesh = plsc.VectorSubcoreMesh(core_axis_name="c", subcore_axis_name="s")

    @functools.partial(
        pl.kernel,
        mesh=mesh,
        out_type=jax.ShapeDtypeStruct((2 * NG, WIDTH), jnp.float32),
        scratch_types=[
            pltpu.VMEM((NCHUNK, CHUNK), jnp.int32),
            pltpu.VMEM((CHUNK, WIDTH), jnp.float32),
            pltpu.VMEM_SHARED((NG, WIDTH), jnp.float32),
        ],
    )
    def scatter(xaug_hbm, asn_hbm, out_hbm, idx_v, chunk_v, table_sh):
        c = lax.axis_index("c")
        s = lax.axis_index("s")
        wid = s * 2 + c
        base = wid * ROWS_PER_W

        zv = jnp.zeros((16,), jnp.float32)

        def zrow(r, carry):
            for k in range(WIDTH // 16):
                chunk_v[r, pl.ds(k * 16, 16)] = zv
            return carry

        lax.fori_loop(0, CHUNK, zrow, 0)
        for q in range(STRIPE // CHUNK):
            pltpu.sync_copy(chunk_v, table_sh.at[pl.ds(s * STRIPE + q * CHUNK, CHUNK)])
        pltpu.sync_copy(asn_hbm.at[pl.ds(wid * NCHUNK, NCHUNK)], idx_v)
        plsc.subcore_barrier()

        for q in range(NCHUNK):
            pltpu.sync_copy(xaug_hbm.at[pl.ds(base + q * CHUNK, CHUNK)], chunk_v)
            pltpu.sync_copy(chunk_v, table_sh.at[idx_v.at[q]], add=True)
        plsc.subcore_barrier()

        for q in range(STRIPE // CHUNK):
            pltpu.sync_copy(table_sh.at[pl.ds(s * STRIPE + q * CHUNK, CHUNK)], chunk_v)
            pltpu.sync_copy(
                chunk_v,
                out_hbm.at[pl.ds(c * NG + s * STRIPE + q * CHUNK, CHUNK)])

    return scatter


def _combine_body(gf_ref, t_ref, out_ref):
    gf = gf_ref[...]
    t = t_ref[0] + t_ref[1]
    sums = t[:, :D]
    counts = t[:, D:D + 1]
    upd = BETA * gf + (1.0 - BETA) * sums / jnp.maximum(counts, 1.0)
    g = jnp.where(counts > 0, upd, gf)
    n = jnp.sqrt(jnp.sum(g * g, axis=1, keepdims=True))
    out_ref[...] = g / jnp.maximum(n, 1e-12)


def _combine(gf, table):
    return pl.pallas_call(
        _combine_body,
        out_shape=jax.ShapeDtypeStruct((NG, D), jnp.float32),
    )(gf, table)


def kernel(x, group_features):
    gfn = _normalize_rows(group_features)
    asn, xaug = _assign(x, gfn)
    asn = asn.reshape(NW * NCHUNK, CHUNK)
    table = _make_scatter()(xaug, asn).reshape(2, NG, WIDTH)
    return _combine(group_features, table)


# speedup vs baseline: 2.0818x; 1.0238x over previous
import functools

import jax
import jax.numpy as jnp
from jax import lax
from jax.experimental import pallas as pl
from jax.experimental.pallas import tpu as pltpu
from jax.experimental.pallas import tpu_sc as plsc

NG = 8192
D = 32
NS = 16384
BETA = 0.99
WIDTH = 128
XB = 512
NW = 32
ROWS_PER_W = NS // NW
CHUNK = 128
NCHUNK = ROWS_PER_W // CHUNK
STRIPE = NG // 16


GB = 128
NGC = NG // GB


def _normalize_body(a_ref, out_ref):
    a = a_ref[...]
    n = jnp.sqrt(jnp.sum(a * a, axis=1, keepdims=True))
    out_ref[...] = a / jnp.maximum(n, 1e-12)


def _normalize_rows(a):
    rows = a.shape[0]
    blk = rows // 4
    return pl.pallas_call(
        _normalize_body,
        grid=(4,),
        in_specs=[pl.BlockSpec((blk, a.shape[1]), lambda i: (i, 0))],
        out_specs=pl.BlockSpec((blk, a.shape[1]), lambda i: (i, 0)),
        out_shape=jax.ShapeDtypeStruct(a.shape, jnp.float32),
    )(a)


def _assign_body(x_ref, gfn_ref, out_ref, xaug_ref):
    x = x_ref[...]
    xaug_ref[...] = jnp.concatenate(
        [x, jnp.ones((XB, 1), jnp.float32),
         jnp.zeros((XB, WIDTH - D - 1), jnp.float32)], axis=1)
    n = jnp.sqrt(jnp.sum(x * x, axis=1, keepdims=True))
    xn = x / jnp.maximum(n, 1e-12)

    m_run = jnp.full((XB, GB), -jnp.inf, jnp.float32)
    c_run = jnp.zeros((XB, GB), jnp.float32)
    for c in range(NGC):
        chunk = lax.dot_general(
            xn, gfn_ref[c * GB:(c + 1) * GB, :], (((1,), (1,)), ((), ())),
            preferred_element_type=jnp.float32)
        better = chunk > m_run
        c_run = jnp.where(better, jnp.float32(c), c_run)
        m_run = jnp.maximum(chunk, m_run)

    m = jnp.max(m_run, axis=-1, keepdims=True)
    lane = lax.broadcasted_iota(jnp.int32, (XB, GB), 1).astype(jnp.float32)
    j = c_run * GB + lane
    loc = jnp.min(jnp.where(m_run == m, j, jnp.float32(NG)), axis=-1)
    out_ref[0] = loc.astype(jnp.int32).reshape(XB // CHUNK, CHUNK)


def _assign(x, gfn):
    grid = NS // XB
    rpb = XB // CHUNK
    return pl.pallas_call(
        _assign_body,
        grid=(grid,),
        in_specs=[
            pl.BlockSpec((XB, D), lambda i: (i, 0)),
            pl.BlockSpec((NG, D), lambda i: (0, 0)),
        ],
        out_specs=[
            pl.BlockSpec((1, rpb, CHUNK), lambda i: (i, 0, 0)),
            pl.BlockSpec((XB, WIDTH), lambda i: (i, 0)),
        ],
        out_shape=[
            jax.ShapeDtypeStruct((grid, rpb, CHUNK), jnp.int32),
            jax.ShapeDtypeStruct((NS, WIDTH), jnp.float32),
        ],
    )(x, gfn)


@functools.lru_cache(maxsize=1)
def _make_scatter():
    mesh = plsc.VectorSubcoreMesh(core_axis_name="c", subcore_axis_name="s")

    @functools.partial(
        pl.kernel,
        mesh=mesh,
        out_type=jax.ShapeDtypeStruct((2 * NG, WIDTH), jnp.float32),
        scratch_types=[
            pltpu.VMEM((NCHUNK, CHUNK), jnp.int32),
            pltpu.VMEM((CHUNK, WIDTH), jnp.float32),
            pltpu.VMEM((CHUNK, WIDTH), jnp.float32),
            pltpu.SemaphoreType.DMA,
            pltpu.SemaphoreType.DMA,
            pltpu.VMEM_SHARED((NG, WIDTH), jnp.float32),
        ],
    )
    def scatter(xaug_hbm, asn_hbm, out_hbm, idx_v, buf0, buf1, sem0, sem1,
                table_sh):
        c = lax.axis_index("c")
        s = lax.axis_index("s")
        wid = s * 2 + c
        base = wid * ROWS_PER_W
        bufs = (buf0, buf1)
        sems = (sem0, sem1)

        zv = jnp.zeros((16,), jnp.float32)

        def zrow(r, carry):
            for k in range(WIDTH // 16):
                buf0[r, pl.ds(k * 16, 16)] = zv
            return carry

        lax.fori_loop(0, CHUNK, zrow, 0)
        for q in range(STRIPE // CHUNK):
            pltpu.sync_copy(buf0, table_sh.at[pl.ds(s * STRIPE + q * CHUNK, CHUNK)])
        pltpu.sync_copy(asn_hbm.at[pl.ds(wid * NCHUNK, NCHUNK)], idx_v)
        plsc.subcore_barrier()

        copies = [None] * NCHUNK
        copies[0] = pltpu.async_copy(
            xaug_hbm.at[pl.ds(base, CHUNK)], bufs[0], sems[0])
        for q in range(NCHUNK):
            copies[q].wait()
            if q + 1 < NCHUNK:
                copies[q + 1] = pltpu.async_copy(
                    xaug_hbm.at[pl.ds(base + (q + 1) * CHUNK, CHUNK)],
                    bufs[(q + 1) % 2], sems[(q + 1) % 2])
            pltpu.sync_copy(bufs[q % 2], table_sh.at[idx_v.at[q]], add=True)
        plsc.subcore_barrier()

        nst = STRIPE // CHUNK
        stores = [None] * nst
        for q in range(nst):
            if q >= 2:
                stores[q - 2].wait()
            pltpu.sync_copy(table_sh.at[pl.ds(s * STRIPE + q * CHUNK, CHUNK)],
                            bufs[q % 2])
            stores[q] = pltpu.async_copy(
                bufs[q % 2],
                out_hbm.at[pl.ds(c * NG + s * STRIPE + q * CHUNK, CHUNK)],
                sems[q % 2])
        for q in range(nst - 2, nst):
            stores[q].wait()

    return scatter


def _combine_body(gf_ref, t_ref, out_ref):
    gf = gf_ref[...]
    t = t_ref[0] + t_ref[1]
    sums = t[:, :D]
    counts = t[:, D:D + 1]
    upd = BETA * gf + (1.0 - BETA) * sums / jnp.maximum(counts, 1.0)
    g = jnp.where(counts > 0, upd, gf)
    n = jnp.sqrt(jnp.sum(g * g, axis=1, keepdims=True))
    out_ref[...] = g / jnp.maximum(n, 1e-12)


def _combine(gf, table):
    nblk = 8
    blk = NG // nblk
    return pl.pallas_call(
        _combine_body,
        grid=(nblk,),
        in_specs=[
            pl.BlockSpec((blk, D), lambda i: (i, 0)),
            pl.BlockSpec((2, blk, WIDTH), lambda i: (0, i, 0)),
        ],
        out_specs=pl.BlockSpec((blk, D), lambda i: (i, 0)),
        out_shape=jax.ShapeDtypeStruct((NG, D), jnp.float32),
    )(gf, table)


def kernel(x, group_features):
    gfn = _normalize_rows(group_features)
    asn, xaug = _assign(x, gfn)
    asn = asn.reshape(NW * NCHUNK, CHUNK)
    table = _make_scatter()(xaug, asn).reshape(2, NG, WIDTH)
    return _combine(group_features, table)


# speedup vs baseline: 2.1166x; 1.0167x over previous
import functools

import jax
import jax.numpy as jnp
from jax import lax
from jax.experimental import pallas as pl
from jax.experimental.pallas import tpu as pltpu
from jax.experimental.pallas import tpu_sc as plsc

NG = 8192
D = 32
NS = 16384
BETA = 0.99
WIDTH = 128
XB = 512
NW = 32
ROWS_PER_W = NS // NW
CHUNK = 128
NCHUNK = ROWS_PER_W // CHUNK
STRIPE = NG // 16


GB = 128
NGC = NG // GB


def _normalize_body(a_ref, out_ref):
    a = a_ref[...]
    n = jnp.sqrt(jnp.sum(a * a, axis=1, keepdims=True))
    out_ref[...] = a / jnp.maximum(n, 1e-12)


def _normalize_rows(a):
    rows = a.shape[0]
    blk = rows // 4
    return pl.pallas_call(
        _normalize_body,
        grid=(4,),
        in_specs=[pl.BlockSpec((blk, a.shape[1]), lambda i: (i, 0))],
        out_specs=pl.BlockSpec((blk, a.shape[1]), lambda i: (i, 0)),
        out_shape=jax.ShapeDtypeStruct(a.shape, jnp.float32),
    )(a)


def _assign_body(x_ref, gf_ref, out_ref, xaug_ref, gfn_ref):
    i = pl.program_id(0)

    @pl.when(i == 0)
    def _():
        gf = gf_ref[...]
        nn = jnp.sqrt(jnp.sum(gf * gf, axis=1, keepdims=True))
        gfn_ref[...] = gf / jnp.maximum(nn, 1e-12)

    x = x_ref[...]
    xaug_ref[...] = jnp.concatenate(
        [x, jnp.ones((XB, 1), jnp.float32),
         jnp.zeros((XB, WIDTH - D - 1), jnp.float32)], axis=1)
    n = jnp.sqrt(jnp.sum(x * x, axis=1, keepdims=True))
    xn = x / jnp.maximum(n, 1e-12)

    m_run = jnp.full((XB, GB), -jnp.inf, jnp.float32)
    c_run = jnp.zeros((XB, GB), jnp.float32)
    for c in range(NGC):
        chunk = lax.dot_general(
            xn, gfn_ref[c * GB:(c + 1) * GB, :], (((1,), (1,)), ((), ())),
            preferred_element_type=jnp.float32)
        better = chunk > m_run
        c_run = jnp.where(better, jnp.float32(c), c_run)
        m_run = jnp.maximum(chunk, m_run)

    m = jnp.max(m_run, axis=-1, keepdims=True)
    lane = lax.broadcasted_iota(jnp.int32, (XB, GB), 1).astype(jnp.float32)
    j = c_run * GB + lane
    loc = jnp.min(jnp.where(m_run == m, j, jnp.float32(NG)), axis=-1)
    out_ref[0] = loc.astype(jnp.int32).reshape(XB // CHUNK, CHUNK)


def _assign(x, gf):
    grid = NS // XB
    rpb = XB // CHUNK
    return pl.pallas_call(
        _assign_body,
        grid=(grid,),
        in_specs=[
            pl.BlockSpec((XB, D), lambda i: (i, 0)),
            pl.BlockSpec((NG, D), lambda i: (0, 0)),
        ],
        out_specs=[
            pl.BlockSpec((1, rpb, CHUNK), lambda i: (i, 0, 0)),
            pl.BlockSpec((XB, WIDTH), lambda i: (i, 0)),
        ],
        out_shape=[
            jax.ShapeDtypeStruct((grid, rpb, CHUNK), jnp.int32),
            jax.ShapeDtypeStruct((NS, WIDTH), jnp.float32),
        ],
        scratch_shapes=[pltpu.VMEM((NG, D), jnp.float32)],
    )(x, gf)


@functools.lru_cache(maxsize=1)
def _make_scatter():
    mesh = plsc.VectorSubcoreMesh(core_axis_name="c", subcore_axis_name="s")

    @functools.partial(
        pl.kernel,
        mesh=mesh,
        out_type=jax.ShapeDtypeStruct((2 * NG, WIDTH), jnp.float32),
        scratch_types=[
            pltpu.VMEM((NCHUNK, CHUNK), jnp.int32),
            pltpu.VMEM((CHUNK, WIDTH), jnp.float32),
            pltpu.VMEM((CHUNK, WIDTH), jnp.float32),
            pltpu.SemaphoreType.DMA,
            pltpu.SemaphoreType.DMA,
            pltpu.VMEM_SHARED((NG, WIDTH), jnp.float32),
        ],
    )
    def scatter(xaug_hbm, asn_hbm, out_hbm, idx_v, buf0, buf1, sem0, sem1,
                table_sh):
        c = lax.axis_index("c")
        s = lax.axis_index("s")
        wid = s * 2 + c
        base = wid * ROWS_PER_W
        bufs = (buf0, buf1)
        sems = (sem0, sem1)

        zv = jnp.zeros((16,), jnp.float32)

        def zrow(r, carry):
            for k in range(WIDTH // 16):
                buf0[r, pl.ds(k * 16, 16)] = zv
            return carry

        lax.fori_loop(0, CHUNK, zrow, 0)
        for q in range(STRIPE // CHUNK):
            pltpu.sync_copy(buf0, table_sh.at[pl.ds(s * STRIPE + q * CHUNK, CHUNK)])
        pltpu.sync_copy(asn_hbm.at[wid], idx_v)
        plsc.subcore_barrier()

        copies = [None] * NCHUNK
        copies[0] = pltpu.async_copy(
            xaug_hbm.at[pl.ds(base, CHUNK)], bufs[0], sems[0])
        for q in range(NCHUNK):
            copies[q].wait()
            if q + 1 < NCHUNK:
                copies[q + 1] = pltpu.async_copy(
                    xaug_hbm.at[pl.ds(base + (q + 1) * CHUNK, CHUNK)],
                    bufs[(q + 1) % 2], sems[(q + 1) % 2])
            pltpu.sync_copy(bufs[q % 2], table_sh.at[idx_v.at[q]], add=True)
        plsc.subcore_barrier()

        nst = STRIPE // CHUNK
        stores = [None] * nst
        for q in range(nst):
            if q >= 2:
                stores[q - 2].wait()
            pltpu.sync_copy(table_sh.at[pl.ds(s * STRIPE + q * CHUNK, CHUNK)],
                            bufs[q % 2])
            stores[q] = pltpu.async_copy(
                bufs[q % 2],
                out_hbm.at[pl.ds(c * NG + s * STRIPE + q * CHUNK, CHUNK)],
                sems[q % 2])
        for q in range(nst - 2, nst):
            stores[q].wait()

    return scatter


def _combine_body(gf_ref, t0_ref, t1_ref, out_ref):
    gf = gf_ref[...]
    t = t0_ref[...] + t1_ref[...]
    sums = t[:, :D]
    counts = t[:, D:D + 1]
    upd = BETA * gf + (1.0 - BETA) * sums / jnp.maximum(counts, 1.0)
    g = jnp.where(counts > 0, upd, gf)
    n = jnp.sqrt(jnp.sum(g * g, axis=1, keepdims=True))
    out_ref[...] = g / jnp.maximum(n, 1e-12)


_COMBINE_NBLK = 8


def _combine(gf, table):
    nblk = _COMBINE_NBLK
    blk = NG // nblk
    return pl.pallas_call(
        _combine_body,
        grid=(nblk,),
        in_specs=[
            pl.BlockSpec((blk, D), lambda i: (i, 0)),
            pl.BlockSpec((blk, WIDTH), lambda i: (i, 0)),
            pl.BlockSpec((blk, WIDTH), lambda i: (i + nblk, 0)),
        ],
        out_specs=pl.BlockSpec((blk, D), lambda i: (i, 0)),
        out_shape=jax.ShapeDtypeStruct((NG, D), jnp.float32),
    )(gf, table, table)


def kernel(x, group_features):
    asn, xaug = _assign(x, group_features)
    table = _make_scatter()(xaug, asn)
    return _combine(group_features, table)
